# Initial kernel scaffold; baseline (speedup 1.0000x reference)
import jax
import jax.numpy as jnp
from jax.experimental import pallas as pl


def kernel(x, edge_index, edge_weight, W1, b1, W2, b2):
    raise NotImplementedError("write your pallas kernel here")


# speedup vs baseline: 8.1898x; 8.1898x over previous
import functools

import jax
import jax.numpy as jnp
from jax import lax
from jax.experimental import pallas as pl
from jax.experimental.pallas import tpu as pltpu
from jax.experimental.pallas import tpu_sc as plsc

N = 10000
E = 320000
D = 128
H = 64
C = 16

NC = 2
NS = 16
NT = NC * NS
K = 128
NCH = 80
EPAD = NT * NCH * K
STRIPE = 624
TAIL = N - NS * STRIPE


def _sc_spmm(feat, srcg, dstg, wg, F):
    mesh = plsc.VectorSubcoreMesh(core_axis_name="c", subcore_axis_name="s")

    @functools.partial(
        pl.kernel,
        out_type=jax.ShapeDtypeStruct((NC * N, F), jnp.float32),
        mesh=mesh,
        compiler_params=pltpu.CompilerParams(use_tc_tiling_on_sc=False),
        scratch_types=[
            pltpu.VMEM((NCH, K), jnp.int32),
            pltpu.VMEM((NCH, K), jnp.int32),
            pltpu.VMEM((NCH, K), jnp.float32),
            pltpu.VMEM((K, F), jnp.float32),
            pltpu.VMEM((K, F), jnp.float32),
            pltpu.VMEM_SHARED((N, F), jnp.float32),
            pltpu.SemaphoreType.DMA,
            pltpu.SemaphoreType.DMA,
        ],
    )
    def spmm(feat_hbm, src_hbm, dst_hbm, w_hbm, out_hbm,
             src_v, dst_v, w_v, rows0, rows1, acc, sem0, sem1):
        c = lax.axis_index("c")
        s = lax.axis_index("s")
        tile = c * NS + s

        pltpu.sync_copy(src_hbm.at[tile], src_v)
        pltpu.sync_copy(dst_hbm.at[tile], dst_v)
        pltpu.sync_copy(w_hbm.at[tile], w_v)

        zero16 = jnp.zeros((16,), jnp.float32)

        def zrow(e, carry):
            for jf in range(F // 16):
                rows0[e, pl.ds(jf * 16, 16)] = zero16
            return carry

        lax.fori_loop(0, K, zrow, 0)
        base = s * STRIPE
        for kk in range(STRIPE // K):
            pltpu.sync_copy(rows0, acc.at[pl.ds(base + kk * K, K)])
        rem = STRIPE % K
        if rem:
            pltpu.sync_copy(rows0.at[pl.ds(0, rem)],
                            acc.at[pl.ds(base + (STRIPE // K) * K, rem)])

        @pl.when(s == NS - 1)
        def _():
            pltpu.sync_copy(rows0.at[pl.ds(0, TAIL)],
                            acc.at[pl.ds(NS * STRIPE, TAIL)])

        plsc.subcore_barrier()

        pltpu.async_copy(feat_hbm.at[src_v.at[0]], rows0, sem0)
        pltpu.async_copy(feat_hbm.at[src_v.at[1]], rows1, sem1)

        dn = lax.GatherDimensionNumbers(
            offset_dims=(), collapsed_slice_dims=(0,), start_index_map=(0,))

        def chunk_step(j, rows, sem):
            pltpu.make_async_copy(feat_hbm.at[src_v.at[j]], rows, sem).wait()

            def gbody(g, carry):
                w16 = w_v[j, pl.ds(g * 16, 16)]
                for e in range(16):
                    we = lax.gather(
                        w16, jnp.full((16, 1), e, jnp.int32), dn,
                        slice_sizes=(1,),
                        mode=lax.GatherScatterMode.PROMISE_IN_BOUNDS)
                    row = g * 16 + e
                    for jf in range(F // 16):
                        sl = pl.ds(jf * 16, 16)
                        rows[row, sl] = rows[row, sl] * we
                return carry

            lax.fori_loop(0, K // 16, gbody, 0)
            pltpu.sync_copy(rows, acc.at[dst_v.at[j]], add=True)

            @pl.when(j + 2 < NCH)
            def _():
                pltpu.async_copy(feat_hbm.at[src_v.at[j + 2]], rows, sem)

        def body(j0, carry):
            chunk_step(2 * j0, rows0, sem0)
            chunk_step(2 * j0 + 1, rows1, sem1)
            return carry

        lax.fori_loop(0, NCH // 2, body, 0)

        plsc.subcore_barrier()
        pltpu.sync_copy(acc.at[pl.ds(base, STRIPE)],
                        out_hbm.at[pl.ds(c * N + base, STRIPE)])

        @pl.when(s == NS - 1)
        def _():
            pltpu.sync_copy(acc.at[pl.ds(NS * STRIPE, TAIL)],
                            out_hbm.at[pl.ds(c * N + NS * STRIPE, TAIL)])

    return spmm(feat, srcg, dstg, wg)


def _tc_matmul1(x, W1):
    BN = 2000

    def body(x_ref, w_ref, o_ref):
        o_ref[...] = jnp.dot(x_ref[...], w_ref[...],
                             preferred_element_type=jnp.float32)

    return pl.pallas_call(
        body,
        out_shape=jax.ShapeDtypeStruct((N, H), jnp.float32),
        grid=(N // BN,),
        in_specs=[
            pl.BlockSpec((BN, D), lambda i: (i, 0)),
            pl.BlockSpec((D, H), lambda i: (0, 0)),
        ],
        out_specs=pl.BlockSpec((BN, H), lambda i: (i, 0)),
    )(x, W1)


def _tc_mid(p, b1, W2):
    BN = 2000

    def body(p_ref, b_ref, w_ref, o_ref):
        h = p_ref[0] + p_ref[1] + b_ref[...]
        h = jnp.maximum(h, 0.0)
        o_ref[...] = jnp.dot(h, w_ref[...], preferred_element_type=jnp.float32)

    return pl.pallas_call(
        body,
        out_shape=jax.ShapeDtypeStruct((N, C), jnp.float32),
        grid=(N // BN,),
        in_specs=[
            pl.BlockSpec((2, BN, H), lambda i: (0, i, 0)),
            pl.BlockSpec((1, H), lambda i: (0, 0)),
            pl.BlockSpec((H, C), lambda i: (0, 0)),
        ],
        out_specs=pl.BlockSpec((BN, C), lambda i: (i, 0)),
    )(p, b1.reshape(1, H), W2)


def _tc_out(p, b2):
    BN = 2000

    def body(p_ref, b_ref, o_ref):
        logits = p_ref[0] + p_ref[1] + b_ref[...]
        m = jnp.max(logits, axis=1, keepdims=True)
        e = jnp.exp(logits - m)
        o_ref[...] = e / jnp.sum(e, axis=1, keepdims=True)

    return pl.pallas_call(
        body,
        out_shape=jax.ShapeDtypeStruct((N, C), jnp.float32),
        grid=(N // BN,),
        in_specs=[
            pl.BlockSpec((2, BN, C), lambda i: (0, i, 0)),
            pl.BlockSpec((1, C), lambda i: (0, 0)),
        ],
        out_specs=pl.BlockSpec((BN, C), lambda i: (i, 0)),
    )(p, b2.reshape(1, C))


def kernel(x, edge_index, edge_weight, W1, b1, W2, b2):
    src = edge_index[0]
    dst = edge_index[1]
    pad = EPAD - E
    srcg = jnp.concatenate([src, jnp.zeros((pad,), jnp.int32)]).reshape(NT, NCH, K)
    dstg = jnp.concatenate([dst, jnp.zeros((pad,), jnp.int32)]).reshape(NT, NCH, K)
    wg = jnp.concatenate([edge_weight, jnp.zeros((pad,), jnp.float32)]).reshape(NT, NCH, K)

    support1 = _tc_matmul1(x, W1)
    p1 = _sc_spmm(support1, srcg, dstg, wg, H)
    support2 = _tc_mid(p1.reshape(NC, N, H), b1, W2)
    p2 = _sc_spmm(support2, srcg, dstg, wg, C)
    return _tc_out(p2.reshape(NC, N, C), b2)


# speedup vs baseline: 9.1552x; 1.1179x over previous
import functools

import jax
import jax.numpy as jnp
from jax import lax
from jax.experimental import pallas as pl
from jax.experimental.pallas import tpu as pltpu
from jax.experimental.pallas import tpu_sc as plsc

N = 10000
E = 320000
D = 128
H = 64
C = 16

NC = 2
NS = 16
NT = NC * NS
K = 128
NCH = 80
NBUF = 4
EPAD = NT * NCH * K
STRIPE = 624
TAIL = N - NS * STRIPE


def _sc_spmm(feat, srcg, dstg, wg, F):
    mesh = plsc.VectorSubcoreMesh(core_axis_name="c", subcore_axis_name="s")

    @functools.partial(
        pl.kernel,
        out_type=jax.ShapeDtypeStruct((NC * N, F), jnp.float32),
        mesh=mesh,
        compiler_params=pltpu.CompilerParams(use_tc_tiling_on_sc=False),
        scratch_types=[
            pltpu.VMEM((NCH, K), jnp.int32),
            pltpu.VMEM((NCH, K), jnp.int32),
            pltpu.VMEM((NCH, K), jnp.float32),
            [pltpu.VMEM((K, F), jnp.float32) for _ in range(NBUF)],
            pltpu.VMEM_SHARED((N, F), jnp.float32),
            [pltpu.SemaphoreType.DMA for _ in range(NBUF)],
            [pltpu.SemaphoreType.DMA for _ in range(NBUF)],
        ],
    )
    def spmm(feat_hbm, src_hbm, dst_hbm, w_hbm, out_hbm,
             src_v, dst_v, w_v, rows, acc, gsem, ssem):
        c = lax.axis_index("c")
        s = lax.axis_index("s")
        tile = c * NS + s

        pltpu.sync_copy(src_hbm.at[tile], src_v)
        pltpu.sync_copy(dst_hbm.at[tile], dst_v)
        pltpu.sync_copy(w_hbm.at[tile], w_v)

        zero16 = jnp.zeros((16,), jnp.float32)

        def zrow(e, carry):
            for jf in range(F // 16):
                rows[0][e, pl.ds(jf * 16, 16)] = zero16
            return carry

        lax.fori_loop(0, K, zrow, 0)
        base = s * STRIPE
        for kk in range(STRIPE // K):
            pltpu.sync_copy(rows[0], acc.at[pl.ds(base + kk * K, K)])
        rem = STRIPE % K
        if rem:
            pltpu.sync_copy(rows[0].at[pl.ds(0, rem)],
                            acc.at[pl.ds(base + (STRIPE // K) * K, rem)])

        @pl.when(s == NS - 1)
        def _():
            pltpu.sync_copy(rows[0].at[pl.ds(0, TAIL)],
                            acc.at[pl.ds(NS * STRIPE, TAIL)])

        plsc.subcore_barrier()

        for b in range(NBUF):
            pltpu.async_copy(feat_hbm.at[src_v.at[b]], rows[b], gsem[b])

        dn = lax.GatherDimensionNumbers(
            offset_dims=(), collapsed_slice_dims=(0,), start_index_map=(0,))

        def chunk_step(i0, b):
            j = i0 * NBUF + b
            pltpu.make_async_copy(
                feat_hbm.at[src_v.at[j]], rows[b], gsem[b]).wait()

            @plsc.parallel_loop(0, K // 16, unroll=2)
            def _(g):
                w16 = w_v[j, pl.ds(g * 16, 16)]
                for e in range(16):
                    we = lax.gather(
                        w16, jnp.full((16, 1), e, jnp.int32), dn,
                        slice_sizes=(1,),
                        mode=lax.GatherScatterMode.PROMISE_IN_BOUNDS)
                    row = g * 16 + e
                    for jf in range(F // 16):
                        sl = pl.ds(jf * 16, 16)
                        rows[b][row, sl] = rows[b][row, sl] * we

            pltpu.async_copy(rows[b], acc.at[dst_v.at[j]], ssem[b], add=True)

            pb = (b - 1) % NBUF
            pj = j - 1

            @pl.when((pj >= 0) & (pj + NBUF < NCH))
            def _():
                pltpu.make_async_copy(
                    rows[pb], acc.at[dst_v.at[pj]], ssem[pb]).wait()
                pltpu.async_copy(
                    feat_hbm.at[src_v.at[pj + NBUF]], rows[pb], gsem[pb])

        def body(i0, carry):
            for b in range(NBUF):
                chunk_step(i0, b)
            return carry

        lax.fori_loop(0, NCH // NBUF, body, 0)

        for b in range(NBUF):
            pltpu.make_async_copy(
                rows[b], acc.at[dst_v.at[NCH - NBUF + b]], ssem[b]).wait()

        plsc.subcore_barrier()
        pltpu.sync_copy(acc.at[pl.ds(base, STRIPE)],
                        out_hbm.at[pl.ds(c * N + base, STRIPE)])

        @pl.when(s == NS - 1)
        def _():
            pltpu.sync_copy(acc.at[pl.ds(NS * STRIPE, TAIL)],
                            out_hbm.at[pl.ds(c * N + NS * STRIPE, TAIL)])

    return spmm(feat, srcg, dstg, wg)


def _tc_matmul1(x, W1):
    BN = 2000

    def body(x_ref, w_ref, o_ref):
        o_ref[...] = jnp.dot(x_ref[...], w_ref[...],
                             preferred_element_type=jnp.float32)

    return pl.pallas_call(
        body,
        out_shape=jax.ShapeDtypeStruct((N, H), jnp.float32),
        grid=(N // BN,),
        in_specs=[
            pl.BlockSpec((BN, D), lambda i: (i, 0)),
            pl.BlockSpec((D, H), lambda i: (0, 0)),
        ],
        out_specs=pl.BlockSpec((BN, H), lambda i: (i, 0)),
    )(x, W1)


def _tc_mid(p, b1, W2):
    BN = 2000

    def body(p_ref, b_ref, w_ref, o_ref):
        h = p_ref[0] + p_ref[1] + b_ref[...]
        h = jnp.maximum(h, 0.0)
        o_ref[...] = jnp.dot(h, w_ref[...], preferred_element_type=jnp.float32)

    return pl.pallas_call(
        body,
        out_shape=jax.ShapeDtypeStruct((N, C), jnp.float32),
        grid=(N // BN,),
        in_specs=[
            pl.BlockSpec((2, BN, H), lambda i: (0, i, 0)),
            pl.BlockSpec((1, H), lambda i: (0, 0)),
            pl.BlockSpec((H, C), lambda i: (0, 0)),
        ],
        out_specs=pl.BlockSpec((BN, C), lambda i: (i, 0)),
    )(p, b1.reshape(1, H), W2)


def _tc_out(p, b2):
    BN = 2000

    def body(p_ref, b_ref, o_ref):
        logits = p_ref[0] + p_ref[1] + b_ref[...]
        m = jnp.max(logits, axis=1, keepdims=True)
        e = jnp.exp(logits - m)
        o_ref[...] = e / jnp.sum(e, axis=1, keepdims=True)

    return pl.pallas_call(
        body,
        out_shape=jax.ShapeDtypeStruct((N, C), jnp.float32),
        grid=(N // BN,),
        in_specs=[
            pl.BlockSpec((2, BN, C), lambda i: (0, i, 0)),
            pl.BlockSpec((1, C), lambda i: (0, 0)),
        ],
        out_specs=pl.BlockSpec((BN, C), lambda i: (i, 0)),
    )(p, b2.reshape(1, C))


def kernel(x, edge_index, edge_weight, W1, b1, W2, b2):
    src = edge_index[0]
    dst = edge_index[1]
    pad = EPAD - E
    srcg = jnp.concatenate([src, jnp.zeros((pad,), jnp.int32)]).reshape(NT, NCH, K)
    dstg = jnp.concatenate([dst, jnp.zeros((pad,), jnp.int32)]).reshape(NT, NCH, K)
    wg = jnp.concatenate([edge_weight, jnp.zeros((pad,), jnp.float32)]).reshape(NT, NCH, K)

    support1 = _tc_matmul1(x, W1)
    p1 = _sc_spmm(support1, srcg, dstg, wg, H)
    support2 = _tc_mid(p1.reshape(NC, N, H), b1, W2)
    p2 = _sc_spmm(support2, srcg, dstg, wg, C)
    return _tc_out(p2.reshape(NC, N, C), b2)


# speedup vs baseline: 9.2559x; 1.0110x over previous
import functools

import jax
import jax.numpy as jnp
from jax import lax
from jax.experimental import pallas as pl
from jax.experimental.pallas import tpu as pltpu
from jax.experimental.pallas import tpu_sc as plsc

N = 10000
E = 320000
D = 128
H = 64
C = 16

NC = 2
NS = 16
NT = NC * NS
K = 128
TOTCH = 2560
TOTCH_PAD = 2688
NBUF = 4
STRIPE = 624
TAIL = N - NS * STRIPE


def _sc_spmm(feat, srcg, dstg, wg, F, ch0, ch1, use_cache):
    mesh = plsc.VectorSubcoreMesh(core_axis_name="c", subcore_axis_name="s")
    chmax = max(ch0, ch1)

    scratch = [
        pltpu.VMEM((chmax, K), jnp.int32),
        pltpu.VMEM((chmax, K), jnp.int32),
        pltpu.VMEM((chmax, K), jnp.float32),
        [pltpu.VMEM((K, F), jnp.float32) for _ in range(NBUF)],
        pltpu.VMEM_SHARED((N, F), jnp.float32),
        [pltpu.SemaphoreType.DMA for _ in range(NBUF)],
        [pltpu.SemaphoreType.DMA for _ in range(NBUF)],
    ]
    if use_cache:
        scratch.append(pltpu.VMEM_SHARED((N, F), jnp.float32))

    @functools.partial(
        pl.kernel,
        out_type=jax.ShapeDtypeStruct((NC * N, F), jnp.float32),
        mesh=mesh,
        compiler_params=pltpu.CompilerParams(use_tc_tiling_on_sc=False),
        scratch_types=scratch,
    )
    def spmm(feat_hbm, src_hbm, dst_hbm, w_hbm, out_hbm,
             src_v, dst_v, w_v, rows, acc, gsem, ssem, *maybe_cache):
        c = lax.axis_index("c")
        s = lax.axis_index("s")
        nch = jnp.where(c == 0, ch0, ch1)
        base_ch = jnp.where(c == 0, s * ch0, NS * ch0 + s * ch1)

        pltpu.sync_copy(src_hbm.at[pl.ds(base_ch, chmax)], src_v)
        pltpu.sync_copy(dst_hbm.at[pl.ds(base_ch, chmax)], dst_v)
        pltpu.sync_copy(w_hbm.at[pl.ds(base_ch, chmax)], w_v)

        base = s * STRIPE
        if use_cache:
            feat_src = maybe_cache[0]
            pltpu.sync_copy(feat_hbm.at[pl.ds(base, STRIPE)],
                            feat_src.at[pl.ds(base, STRIPE)])

            @pl.when(s == NS - 1)
            def _():
                pltpu.sync_copy(feat_hbm.at[pl.ds(NS * STRIPE, TAIL)],
                                feat_src.at[pl.ds(NS * STRIPE, TAIL)])
        else:
            feat_src = feat_hbm

        zero16 = jnp.zeros((16,), jnp.float32)

        def zrow(e, carry):
            for jf in range(F // 16):
                rows[0][e, pl.ds(jf * 16, 16)] = zero16
            return carry

        lax.fori_loop(0, K, zrow, 0)
        for kk in range(STRIPE // K):
            pltpu.sync_copy(rows[0], acc.at[pl.ds(base + kk * K, K)])
        rem = STRIPE % K
        if rem:
            pltpu.sync_copy(rows[0].at[pl.ds(0, rem)],
                            acc.at[pl.ds(base + (STRIPE // K) * K, rem)])

        @pl.when(s == NS - 1)
        def _():
            pltpu.sync_copy(rows[0].at[pl.ds(0, TAIL)],
                            acc.at[pl.ds(NS * STRIPE, TAIL)])

        plsc.subcore_barrier()

        for b in range(NBUF):
            pltpu.async_copy(feat_src.at[src_v.at[b]], rows[b], gsem[b])

        dn = lax.GatherDimensionNumbers(
            offset_dims=(), collapsed_slice_dims=(0,), start_index_map=(0,))

        def chunk_step(i0, b):
            j = i0 * NBUF + b
            pltpu.make_async_copy(
                feat_src.at[src_v.at[j]], rows[b], gsem[b]).wait()

            @plsc.parallel_loop(0, K // 16, unroll=2)
            def _(g):
                w16 = w_v[j, pl.ds(g * 16, 16)]
                for e in range(16):
                    we = lax.gather(
                        w16, jnp.full((16, 1), e, jnp.int32), dn,
                        slice_sizes=(1,),
                        mode=lax.GatherScatterMode.PROMISE_IN_BOUNDS)
                    row = g * 16 + e
                    for jf in range(F // 16):
                        sl = pl.ds(jf * 16, 16)
                        rows[b][row, sl] = rows[b][row, sl] * we

            pltpu.async_copy(rows[b], acc.at[dst_v.at[j]], ssem[b], add=True)

            pb = (b - 1) % NBUF
            pj = j - 1

            @pl.when((pj >= 0) & (pj + NBUF < nch))
            def _():
                pltpu.make_async_copy(
                    rows[pb], acc.at[dst_v.at[pj]], ssem[pb]).wait()
                pltpu.async_copy(
                    feat_src.at[src_v.at[pj + NBUF]], rows[pb], gsem[pb])

        def body(i0, carry):
            for b in range(NBUF):
                chunk_step(i0, b)
            return carry

        lax.fori_loop(0, nch // NBUF, body, 0)

        for b in range(NBUF):
            pltpu.make_async_copy(
                rows[b], acc.at[dst_v.at[nch - NBUF + b]], ssem[b]).wait()

        plsc.subcore_barrier()
        pltpu.sync_copy(acc.at[pl.ds(base, STRIPE)],
                        out_hbm.at[pl.ds(c * N + base, STRIPE)])

        @pl.when(s == NS - 1)
        def _():
            pltpu.sync_copy(acc.at[pl.ds(NS * STRIPE, TAIL)],
                            out_hbm.at[pl.ds(c * N + NS * STRIPE, TAIL)])

    return spmm(feat, srcg, dstg, wg)


def _tc_matmul1(x, W1):
    BN = 2000

    def body(x_ref, w_ref, o_ref):
        o_ref[...] = jnp.dot(x_ref[...], w_ref[...],
                             preferred_element_type=jnp.float32)

    return pl.pallas_call(
        body,
        out_shape=jax.ShapeDtypeStruct((N, H), jnp.float32),
        grid=(N // BN,),
        in_specs=[
            pl.BlockSpec((BN, D), lambda i: (i, 0)),
            pl.BlockSpec((D, H), lambda i: (0, 0)),
        ],
        out_specs=pl.BlockSpec((BN, H), lambda i: (i, 0)),
    )(x, W1)


def _tc_mid(p, b1, W2):
    BN = 2000

    def body(p_ref, b_ref, w_ref, o_ref):
        h = p_ref[0] + p_ref[1] + b_ref[...]
        h = jnp.maximum(h, 0.0)
        o_ref[...] = jnp.dot(h, w_ref[...], preferred_element_type=jnp.float32)

    return pl.pallas_call(
        body,
        out_shape=jax.ShapeDtypeStruct((N, C), jnp.float32),
        grid=(N // BN,),
        in_specs=[
            pl.BlockSpec((2, BN, H), lambda i: (0, i, 0)),
            pl.BlockSpec((1, H), lambda i: (0, 0)),
            pl.BlockSpec((H, C), lambda i: (0, 0)),
        ],
        out_specs=pl.BlockSpec((BN, C), lambda i: (i, 0)),
    )(p, b1.reshape(1, H), W2)


def _tc_out(p, b2):
    BN = 2000

    def body(p_ref, b_ref, o_ref):
        logits = p_ref[0] + p_ref[1] + b_ref[...]
        m = jnp.max(logits, axis=1, keepdims=True)
        e = jnp.exp(logits - m)
        o_ref[...] = e / jnp.sum(e, axis=1, keepdims=True)

    return pl.pallas_call(
        body,
        out_shape=jax.ShapeDtypeStruct((N, C), jnp.float32),
        grid=(N // BN,),
        in_specs=[
            pl.BlockSpec((2, BN, C), lambda i: (0, i, 0)),
            pl.BlockSpec((1, C), lambda i: (0, 0)),
        ],
        out_specs=pl.BlockSpec((BN, C), lambda i: (i, 0)),
    )(p, b2.reshape(1, C))


def kernel(x, edge_index, edge_weight, W1, b1, W2, b2):
    src = edge_index[0]
    dst = edge_index[1]
    pad = TOTCH_PAD * K - E
    srcg = jnp.concatenate([src, jnp.zeros((pad,), jnp.int32)]).reshape(TOTCH_PAD, K)
    dstg = jnp.concatenate([dst, jnp.zeros((pad,), jnp.int32)]).reshape(TOTCH_PAD, K)
    wg = jnp.concatenate([edge_weight, jnp.zeros((pad,), jnp.float32)]).reshape(TOTCH_PAD, K)

    support1 = _tc_matmul1(x, W1)
    p1 = _sc_spmm(support1, srcg, dstg, wg, H,
                  ch0=120, ch1=40, use_cache=False)
    support2 = _tc_mid(p1.reshape(NC, N, H), b1, W2)
    p2 = _sc_spmm(support2, srcg, dstg, wg, C,
                  ch0=80, ch1=80, use_cache=True)
    return _tc_out(p2.reshape(NC, N, C), b2)


# speedup vs baseline: 15.9480x; 1.7230x over previous
import functools

import jax
import jax.numpy as jnp
from jax import lax
from jax.experimental import pallas as pl
from jax.experimental.pallas import tpu as pltpu
from jax.experimental.pallas import tpu_sc as plsc

N = 10000
E = 320000
D = 128
H = 64
C = 16

NC = 2
NS = 16
NT = NC * NS
K = 128
TOTCH = 2560
TOTCH_PAD = 2688
NBUF = 4
STRIPE = 624
TAIL = N - NS * STRIPE


def _sc_spmm(feat, srcg, dstg, wg, F, fsplit):
    mesh = plsc.VectorSubcoreMesh(core_axis_name="c", subcore_axis_name="s")
    chmax = TOTCH // NS if fsplit else TOTCH // NT

    scratch = [
        pltpu.VMEM((chmax, K), jnp.int32),
        pltpu.VMEM((chmax, K), jnp.int32),
        pltpu.VMEM((chmax, K), jnp.float32),
        [pltpu.VMEM((K, F), jnp.float32) for _ in range(NBUF)],
        pltpu.VMEM_SHARED((N, F), jnp.float32),
        pltpu.VMEM_SHARED((N, F), jnp.float32),
        [pltpu.SemaphoreType.DMA for _ in range(NBUF)],
        [pltpu.SemaphoreType.DMA for _ in range(NBUF)],
    ]

    @functools.partial(
        pl.kernel,
        out_type=jax.ShapeDtypeStruct((NC * N, F), jnp.float32),
        mesh=mesh,
        compiler_params=pltpu.CompilerParams(use_tc_tiling_on_sc=False),
        scratch_types=scratch,
    )
    def spmm(feat_hbm, src_hbm, dst_hbm, w_hbm, out_hbm,
             src_v, dst_v, w_v, rows, acc, feat_src, gsem, ssem):
        c = lax.axis_index("c")
        s = lax.axis_index("s")
        nch = chmax
        if fsplit:
            base_ch = s * chmax
            frow = c * N
        else:
            base_ch = (c * NS + s) * chmax
            frow = 0

        pltpu.sync_copy(src_hbm.at[pl.ds(base_ch, chmax)], src_v)
        pltpu.sync_copy(dst_hbm.at[pl.ds(base_ch, chmax)], dst_v)
        pltpu.sync_copy(w_hbm.at[pl.ds(base_ch, chmax)], w_v)

        base = s * STRIPE
        pltpu.sync_copy(feat_hbm.at[pl.ds(frow + base, STRIPE)],
                        feat_src.at[pl.ds(base, STRIPE)])

        @pl.when(s == NS - 1)
        def _():
            pltpu.sync_copy(feat_hbm.at[pl.ds(frow + NS * STRIPE, TAIL)],
                            feat_src.at[pl.ds(NS * STRIPE, TAIL)])

        zero16 = jnp.zeros((16,), jnp.float32)

        def zrow(e, carry):
            for jf in range(F // 16):
                rows[0][e, pl.ds(jf * 16, 16)] = zero16
            return carry

        lax.fori_loop(0, K, zrow, 0)
        for kk in range(STRIPE // K):
            pltpu.sync_copy(rows[0], acc.at[pl.ds(base + kk * K, K)])
        rem = STRIPE % K
        if rem:
            pltpu.sync_copy(rows[0].at[pl.ds(0, rem)],
                            acc.at[pl.ds(base + (STRIPE // K) * K, rem)])

        @pl.when(s == NS - 1)
        def _():
            pltpu.sync_copy(rows[0].at[pl.ds(0, TAIL)],
                            acc.at[pl.ds(NS * STRIPE, TAIL)])

        plsc.subcore_barrier()

        for b in range(NBUF):
            pltpu.async_copy(feat_src.at[src_v.at[b]], rows[b], gsem[b])

        dn = lax.GatherDimensionNumbers(
            offset_dims=(), collapsed_slice_dims=(0,), start_index_map=(0,))

        def chunk_step(i0, b):
            j = i0 * NBUF + b
            pltpu.make_async_copy(
                feat_src.at[src_v.at[j]], rows[b], gsem[b]).wait()

            @plsc.parallel_loop(0, K // 16, unroll=2)
            def _(g):
                w16 = w_v[j, pl.ds(g * 16, 16)]
                for e in range(16):
                    we = lax.gather(
                        w16, jnp.full((16, 1), e, jnp.int32), dn,
                        slice_sizes=(1,),
                        mode=lax.GatherScatterMode.PROMISE_IN_BOUNDS)
                    row = g * 16 + e
                    for jf in range(F // 16):
                        sl = pl.ds(jf * 16, 16)
                        rows[b][row, sl] = rows[b][row, sl] * we

            pltpu.async_copy(rows[b], acc.at[dst_v.at[j]], ssem[b], add=True)

            pb = (b - 1) % NBUF
            pj = j - 1

            @pl.when((pj >= 0) & (pj + NBUF < nch))
            def _():
                pltpu.make_async_copy(
                    rows[pb], acc.at[dst_v.at[pj]], ssem[pb]).wait()
                pltpu.async_copy(
                    feat_src.at[src_v.at[pj + NBUF]], rows[pb], gsem[pb])

        def body(i0, carry):
            for b in range(NBUF):
                chunk_step(i0, b)
            return carry

        lax.fori_loop(0, nch // NBUF, body, 0)

        for b in range(NBUF):
            pltpu.make_async_copy(
                rows[b], acc.at[dst_v.at[nch - NBUF + b]], ssem[b]).wait()

        plsc.subcore_barrier()
        pltpu.sync_copy(acc.at[pl.ds(base, STRIPE)],
                        out_hbm.at[pl.ds(c * N + base, STRIPE)])

        @pl.when(s == NS - 1)
        def _():
            pltpu.sync_copy(acc.at[pl.ds(NS * STRIPE, TAIL)],
                            out_hbm.at[pl.ds(c * N + NS * STRIPE, TAIL)])

    return spmm(feat, srcg, dstg, wg)


def _tc_matmul1(x, W1):
    BN = 2000
    H2 = H // 2

    def body(x_ref, w_ref, o_ref):
        r = jnp.dot(x_ref[...], w_ref[...], preferred_element_type=jnp.float32)
        o_ref[0] = r[:, :H2]
        o_ref[1] = r[:, H2:]

    return pl.pallas_call(
        body,
        out_shape=jax.ShapeDtypeStruct((NC, N, H2), jnp.float32),
        grid=(N // BN,),
        in_specs=[
            pl.BlockSpec((BN, D), lambda i: (i, 0)),
            pl.BlockSpec((D, H), lambda i: (0, 0)),
        ],
        out_specs=pl.BlockSpec((NC, BN, H2), lambda i: (0, i, 0)),
    )(x, W1)


def _tc_mid(p, b1, W2):
    BN = 2000
    H2 = H // 2

    def body(p_ref, b_ref, w_ref, o_ref):
        h = jnp.concatenate([p_ref[0], p_ref[1]], axis=1) + b_ref[...]
        h = jnp.maximum(h, 0.0)
        o_ref[...] = jnp.dot(h, w_ref[...], preferred_element_type=jnp.float32)

    return pl.pallas_call(
        body,
        out_shape=jax.ShapeDtypeStruct((N, C), jnp.float32),
        grid=(N // BN,),
        in_specs=[
            pl.BlockSpec((2, BN, H2), lambda i: (0, i, 0)),
            pl.BlockSpec((1, H), lambda i: (0, 0)),
            pl.BlockSpec((H, C), lambda i: (0, 0)),
        ],
        out_specs=pl.BlockSpec((BN, C), lambda i: (i, 0)),
    )(p, b1.reshape(1, H), W2)


def _tc_out(p, b2):
    BN = 2000

    def body(p_ref, b_ref, o_ref):
        logits = p_ref[0] + p_ref[1] + b_ref[...]
        m = jnp.max(logits, axis=1, keepdims=True)
        e = jnp.exp(logits - m)
        o_ref[...] = e / jnp.sum(e, axis=1, keepdims=True)

    return pl.pallas_call(
        body,
        out_shape=jax.ShapeDtypeStruct((N, C), jnp.float32),
        grid=(N // BN,),
        in_specs=[
            pl.BlockSpec((2, BN, C), lambda i: (0, i, 0)),
            pl.BlockSpec((1, C), lambda i: (0, 0)),
        ],
        out_specs=pl.BlockSpec((BN, C), lambda i: (i, 0)),
    )(p, b2.reshape(1, C))


def kernel(x, edge_index, edge_weight, W1, b1, W2, b2):
    src = edge_index[0]
    dst = edge_index[1]
    pad = TOTCH_PAD * K - E
    srcg = jnp.concatenate([src, jnp.zeros((pad,), jnp.int32)]).reshape(TOTCH_PAD, K)
    dstg = jnp.concatenate([dst, jnp.zeros((pad,), jnp.int32)]).reshape(TOTCH_PAD, K)
    wg = jnp.concatenate([edge_weight, jnp.zeros((pad,), jnp.float32)]).reshape(TOTCH_PAD, K)

    support1 = _tc_matmul1(x, W1)
    p1 = _sc_spmm(support1.reshape(NC * N, H // 2), srcg, dstg, wg,
                  H // 2, fsplit=True)
    support2 = _tc_mid(p1.reshape(NC, N, H // 2), b1, W2)
    p2 = _sc_spmm(support2, srcg, dstg, wg, C, fsplit=False)
    return _tc_out(p2.reshape(NC, N, C), b2)


# speedup vs baseline: 16.9346x; 1.0619x over previous
import functools

import jax
import jax.numpy as jnp
from jax import lax
from jax.experimental import pallas as pl
from jax.experimental.pallas import tpu as pltpu
from jax.experimental.pallas import tpu_sc as plsc

N = 10000
E = 320000
D = 128
H = 64
C = 16

NC = 2
NS = 16
NT = NC * NS
K = 128
TOTCH = 2560
TOTCH_PAD = 2560
NBUF = 4
STRIPE = 624
TAIL = N - NS * STRIPE


def _sc_spmm(feat, eig, wg, F, fsplit):
    mesh = plsc.VectorSubcoreMesh(core_axis_name="c", subcore_axis_name="s")
    chmax = TOTCH // NS if fsplit else TOTCH // NT

    scratch = [
        pltpu.VMEM((chmax, K), jnp.int32),
        pltpu.VMEM((chmax, K), jnp.int32),
        pltpu.VMEM((chmax, K), jnp.float32),
        [pltpu.VMEM((K, F), jnp.float32) for _ in range(NBUF)],
        pltpu.VMEM_SHARED((N, F), jnp.float32),
        pltpu.VMEM_SHARED((N, F), jnp.float32),
        [pltpu.SemaphoreType.DMA for _ in range(NBUF)],
        [pltpu.SemaphoreType.DMA for _ in range(NBUF)],
    ]

    @functools.partial(
        pl.kernel,
        out_type=jax.ShapeDtypeStruct((NC, N, F), jnp.float32),
        mesh=mesh,
        compiler_params=pltpu.CompilerParams(use_tc_tiling_on_sc=False),
        scratch_types=scratch,
    )
    def spmm(feat_hbm, ei_hbm, w_hbm, out_hbm,
             src_v, dst_v, w_v, rows, acc, feat_src, gsem, ssem):
        c = lax.axis_index("c")
        s = lax.axis_index("s")
        nch = chmax
        base_ch = s * chmax if fsplit else (c * NS + s) * chmax

        pltpu.sync_copy(ei_hbm.at[0, pl.ds(base_ch, chmax)], src_v)
        pltpu.sync_copy(ei_hbm.at[1, pl.ds(base_ch, chmax)], dst_v)
        pltpu.sync_copy(w_hbm.at[pl.ds(base_ch, chmax)], w_v)

        base = s * STRIPE
        fc = c if fsplit else 0
        pltpu.sync_copy(feat_hbm.at[fc, pl.ds(base, STRIPE)],
                        feat_src.at[pl.ds(base, STRIPE)])

        @pl.when(s == NS - 1)
        def _():
            pltpu.sync_copy(feat_hbm.at[fc, pl.ds(NS * STRIPE, TAIL)],
                            feat_src.at[pl.ds(NS * STRIPE, TAIL)])

        zero16 = jnp.zeros((16,), jnp.float32)

        def zrow(e, carry):
            for jf in range(F // 16):
                rows[0][e, pl.ds(jf * 16, 16)] = zero16
            return carry

        lax.fori_loop(0, K, zrow, 0)
        for kk in range(STRIPE // K):
            pltpu.sync_copy(rows[0], acc.at[pl.ds(base + kk * K, K)])
        rem = STRIPE % K
        if rem:
            pltpu.sync_copy(rows[0].at[pl.ds(0, rem)],
                            acc.at[pl.ds(base + (STRIPE // K) * K, rem)])

        @pl.when(s == NS - 1)
        def _():
            pltpu.sync_copy(rows[0].at[pl.ds(0, TAIL)],
                            acc.at[pl.ds(NS * STRIPE, TAIL)])

        plsc.subcore_barrier()

        for b in range(NBUF):
            pltpu.async_copy(feat_src.at[src_v.at[b]], rows[b], gsem[b])

        dn = lax.GatherDimensionNumbers(
            offset_dims=(), collapsed_slice_dims=(0,), start_index_map=(0,))

        def chunk_step(i0, b):
            j = i0 * NBUF + b
            pltpu.make_async_copy(
                feat_src.at[src_v.at[j]], rows[b], gsem[b]).wait()

            @plsc.parallel_loop(0, K // 16, unroll=2)
            def _(g):
                w16 = w_v[j, pl.ds(g * 16, 16)]
                for e in range(16):
                    we = lax.gather(
                        w16, jnp.full((16, 1), e, jnp.int32), dn,
                        slice_sizes=(1,),
                        mode=lax.GatherScatterMode.PROMISE_IN_BOUNDS)
                    row = g * 16 + e
                    for jf in range(F // 16):
                        sl = pl.ds(jf * 16, 16)
                        rows[b][row, sl] = rows[b][row, sl] * we

            pltpu.async_copy(rows[b], acc.at[dst_v.at[j]], ssem[b], add=True)

            pb = (b - 1) % NBUF
            pj = j - 1

            @pl.when((pj >= 0) & (pj + NBUF < nch))
            def _():
                pltpu.make_async_copy(
                    rows[pb], acc.at[dst_v.at[pj]], ssem[pb]).wait()
                pltpu.async_copy(
                    feat_src.at[src_v.at[pj + NBUF]], rows[pb], gsem[pb])

        def body(i0, carry):
            for b in range(NBUF):
                chunk_step(i0, b)
            return carry

        lax.fori_loop(0, nch // NBUF, body, 0)

        for b in range(NBUF):
            pltpu.make_async_copy(
                rows[b], acc.at[dst_v.at[nch - NBUF + b]], ssem[b]).wait()

        plsc.subcore_barrier()
        pltpu.sync_copy(acc.at[pl.ds(base, STRIPE)],
                        out_hbm.at[c, pl.ds(base, STRIPE)])

        @pl.when(s == NS - 1)
        def _():
            pltpu.sync_copy(acc.at[pl.ds(NS * STRIPE, TAIL)],
                            out_hbm.at[c, pl.ds(NS * STRIPE, TAIL)])

    return spmm(feat, eig, wg)


def _tc_matmul1(x, W1):
    BN = 2000
    H2 = H // 2

    def body(x_ref, w_ref, o_ref):
        r = jnp.dot(x_ref[...], w_ref[...], preferred_element_type=jnp.float32)
        o_ref[0] = r[:, :H2]
        o_ref[1] = r[:, H2:]

    return pl.pallas_call(
        body,
        out_shape=jax.ShapeDtypeStruct((NC, N, H2), jnp.float32),
        grid=(N // BN,),
        in_specs=[
            pl.BlockSpec((BN, D), lambda i: (i, 0)),
            pl.BlockSpec((D, H), lambda i: (0, 0)),
        ],
        out_specs=pl.BlockSpec((NC, BN, H2), lambda i: (0, i, 0)),
    )(x, W1)


def _tc_mid(p, b1, W2):
    BN = 2000
    H2 = H // 2

    def body(p_ref, b_ref, w_ref, o_ref):
        h = jnp.concatenate([p_ref[0], p_ref[1]], axis=1) + b_ref[...]
        h = jnp.maximum(h, 0.0)
        o_ref[0] = jnp.dot(h, w_ref[...], preferred_element_type=jnp.float32)

    return pl.pallas_call(
        body,
        out_shape=jax.ShapeDtypeStruct((1, N, C), jnp.float32),
        grid=(N // BN,),
        in_specs=[
            pl.BlockSpec((2, BN, H2), lambda i: (0, i, 0)),
            pl.BlockSpec((1, H), lambda i: (0, 0)),
            pl.BlockSpec((H, C), lambda i: (0, 0)),
        ],
        out_specs=pl.BlockSpec((1, BN, C), lambda i: (0, i, 0)),
    )(p, b1.reshape(1, H), W2)


def _tc_out(p, b2):
    BN = 2000

    def body(p_ref, b_ref, o_ref):
        logits = p_ref[0] + p_ref[1] + b_ref[...]
        m = jnp.max(logits, axis=1, keepdims=True)
        e = jnp.exp(logits - m)
        o_ref[...] = e / jnp.sum(e, axis=1, keepdims=True)

    return pl.pallas_call(
        body,
        out_shape=jax.ShapeDtypeStruct((N, C), jnp.float32),
        grid=(N // BN,),
        in_specs=[
            pl.BlockSpec((2, BN, C), lambda i: (0, i, 0)),
            pl.BlockSpec((1, C), lambda i: (0, 0)),
        ],
        out_specs=pl.BlockSpec((BN, C), lambda i: (i, 0)),
    )(p, b2.reshape(1, C))


def kernel(x, edge_index, edge_weight, W1, b1, W2, b2):
    pad = TOTCH_PAD * K - E
    eig = jnp.pad(edge_index, ((0, 0), (0, pad))).reshape(2, TOTCH_PAD, K)
    wg = jnp.pad(edge_weight, (0, pad)).reshape(TOTCH_PAD, K)

    support1 = _tc_matmul1(x, W1)
    p1 = _sc_spmm(support1, eig, wg, H // 2, fsplit=True)
    support2 = _tc_mid(p1, b1, W2)
    p2 = _sc_spmm(support2, eig, wg, C, fsplit=False)
    return _tc_out(p2, b2)
